# baseline (speedup 1.0000x reference)
import functools

import jax
import jax.numpy as jnp
from jax import lax
from jax.experimental import pallas as pl
from jax.experimental.pallas import tpu as pltpu
from jax.experimental.pallas import tpu_sc as plsc

NC = 2
NS = 16
NW = NC * NS
K = 64
NB = 4


def _round_up(a, m):
    return (a + m - 1) // m * m


def _sc_mesh():
    return plsc.VectorSubcoreMesh(
        core_axis_name="c", subcore_axis_name="s", num_cores=NC, num_subcores=NS
    )


def _make_deg_kernel(cpt, npad, d):
    rpt = npad // NS

    def body(dst_hbm, zeros_hbm, ones_hbm, out_hbm, dst_v, ones_v, acc, sem):
        cid = lax.axis_index("c")
        sid = lax.axis_index("s")
        wid = sid * NC + cid
        pltpu.sync_copy(zeros_hbm.at[pl.ds(sid * rpt, rpt)],
                        acc.at[pl.ds(sid * rpt, rpt)])
        pltpu.sync_copy(ones_hbm, ones_v)
        pltpu.sync_copy(dst_hbm.at[wid], dst_v)
        plsc.subcore_barrier()

        def group(g, carry):
            descs = [
                pltpu.async_copy(ones_v, acc.at[dst_v.at[4 * g + b]], sem,
                                 add=True)
                for b in range(4)
            ]
            for desc in descs:
                desc.wait()
            return carry

        lax.fori_loop(0, cpt // 4, group, 0)

        def chunk(j, carry):
            pltpu.sync_copy(ones_v, acc.at[dst_v.at[j]], add=True)
            return carry

        lax.fori_loop(cpt // 4 * 4, cpt, chunk, 0)
        plsc.subcore_barrier()
        pltpu.sync_copy(acc.at[pl.ds(sid * rpt, rpt)],
                        out_hbm.at[cid].at[pl.ds(sid * rpt, rpt)])

    return pl.kernel(
        body,
        out_type=jax.ShapeDtypeStruct((NC, npad, d), jnp.float32),
        mesh=_sc_mesh(),
        scratch_types=[
            pltpu.VMEM((cpt, K), jnp.int32),
            pltpu.VMEM((K, d), jnp.float32),
            pltpu.VMEM_SHARED((npad, d), jnp.float32),
            pltpu.SemaphoreType.DMA,
        ],
    )


def _make_agg_kernel(cpt, npad, d):
    rpt = npad // NS
    assert cpt % (4 * NB) == 0

    def body(h_hbm, src_hbm, dst_hbm, zeros_hbm, out_hbm,
             src_v, dst_v, r0, r1, r2, r3, acc,
             g0, g1, g2, g3, s0, s1, s2, s3):
        rows = [r0, r1, r2, r3]
        gs = [g0, g1, g2, g3]
        ss = [s0, s1, s2, s3]
        cid = lax.axis_index("c")
        sid = lax.axis_index("s")
        wid = sid * NC + cid
        pltpu.sync_copy(zeros_hbm.at[pl.ds(sid * rpt, rpt)],
                        acc.at[pl.ds(sid * rpt, rpt)])
        plsc.subcore_barrier()

        hcpt = cpt // 4
        for half in range(4):
            pltpu.sync_copy(src_hbm.at[wid].at[pl.ds(half * hcpt, hcpt)], src_v)
            pltpu.sync_copy(dst_hbm.at[wid].at[pl.ds(half * hcpt, hcpt)], dst_v)
            for b in range(NB):
                pltpu.async_copy(h_hbm.at[src_v.at[b]], rows[b], gs[b])

            def group(g, carry):
                sds = []
                for b in range(NB):
                    j = NB * g + b
                    pltpu.make_async_copy(h_hbm.at[src_v.at[j]], rows[b],
                                          gs[b]).wait()
                    sds.append(pltpu.async_copy(rows[b], acc.at[dst_v.at[j]],
                                                ss[b], add=True))
                for b in range(NB):
                    jn = jnp.minimum(NB * g + b + NB, hcpt - 1)
                    sds[b].wait()
                    pltpu.async_copy(h_hbm.at[src_v.at[jn]], rows[b], gs[b])
                return carry

            lax.fori_loop(0, hcpt // NB, group, 0)
            for b in range(NB):
                pltpu.make_async_copy(h_hbm.at[src_v.at[hcpt - 1]], rows[b],
                                      gs[b]).wait()
        plsc.subcore_barrier()
        pltpu.sync_copy(acc.at[pl.ds(sid * rpt, rpt)],
                        out_hbm.at[cid].at[pl.ds(sid * rpt, rpt)])

    return pl.kernel(
        body,
        out_type=jax.ShapeDtypeStruct((NC, npad, d), jnp.float32),
        mesh=_sc_mesh(),
        scratch_types=(
            [pltpu.VMEM((cpt // 4, K), jnp.int32),
             pltpu.VMEM((cpt // 4, K), jnp.int32)]
            + [pltpu.VMEM((K, d), jnp.float32) for _ in range(NB)]
            + [pltpu.VMEM_SHARED((npad, d), jnp.float32)]
            + [pltpu.SemaphoreType.DMA for _ in range(2 * NB)]
        ),
    )


def _tc_grid_specs(n, bn, d, npad, n_deg, n_s):
    del npad
    grid = n // bn
    s_spec = [pl.BlockSpec((1, bn, d), (lambda i, j=j: (j, i, 0)))
              for j in range(n_s)]
    d_spec = [pl.BlockSpec((bn, 8), (lambda i: (i, 0)))
              for _ in range(n_deg)]
    return grid, s_spec, d_spec


def _dinv_kernel(npad, d):
    bnd = npad // 8

    def body(d0, d1, o):
        dv = lax.rsqrt(1.0 + d0[0, :, 0:1] + d1[0, :, 0:1])
        o[...] = jnp.broadcast_to(dv, (bnd, 8))

    return pl.pallas_call(
        body,
        grid=8,
        in_specs=[
            pl.BlockSpec((1, bnd, d), lambda i: (0, i, 0)),
            pl.BlockSpec((1, bnd, d), lambda i: (1, i, 0)),
        ],
        out_specs=pl.BlockSpec((bnd, 8), lambda i: (i, 0)),
        out_shape=jax.ShapeDtypeStruct((npad, 8), jnp.float32),
    )


def _matmul(n, bn, d):
    def body(x, w, o):
        o[...] = jnp.dot(x[...], w[...], preferred_element_type=jnp.float32)

    return pl.pallas_call(
        body,
        grid=n // bn,
        in_specs=[
            pl.BlockSpec((bn, d), lambda i: (i, 0)),
            pl.BlockSpec((d, d), lambda i: (0, 0)),
        ],
        out_specs=pl.BlockSpec((bn, d), lambda i: (i, 0)),
        out_shape=jax.ShapeDtypeStruct((n, d), jnp.float32),
    )


def _scale(n, bn, d):
    grid, _, d_spec = _tc_grid_specs(n, bn, d, None, 1, 0)

    def body(dv, t, o):
        o[...] = t[...] * dv[:, 0:1]

    return pl.pallas_call(
        body,
        grid=grid,
        in_specs=d_spec + [pl.BlockSpec((bn, d), lambda i: (i, 0))],
        out_specs=pl.BlockSpec((bn, d), lambda i: (i, 0)),
        out_shape=jax.ShapeDtypeStruct((n, d), jnp.float32),
    )


def _mid_layer(n, bn, d):
    grid, s_spec, d_spec = _tc_grid_specs(n, bn, d, None, 1, 2)

    def body(s0, s1, hp, dv, b1, w2, o):
        dinv = dv[:, 0:1]
        h = dinv * (s0[0] + s1[0] + hp[...]) + b1[...]
        h = jnp.maximum(h, 0.0)
        o[...] = dinv * jnp.dot(h, w2[...], preferred_element_type=jnp.float32)

    return pl.pallas_call(
        body,
        grid=grid,
        in_specs=s_spec + [pl.BlockSpec((bn, d), lambda i: (i, 0))] + d_spec + [
            pl.BlockSpec((1, d), lambda i: (0, 0)),
            pl.BlockSpec((d, d), lambda i: (0, 0)),
        ],
        out_specs=pl.BlockSpec((bn, d), lambda i: (i, 0)),
        out_shape=jax.ShapeDtypeStruct((n, d), jnp.float32),
    )


def _final_layer(n, bn, d):
    grid, s_spec, d_spec = _tc_grid_specs(n, bn, d, None, 1, 2)

    def body(s0, s1, gp, dv, b2, wfc, bfc, o):
        g = dv[:, 0:1] * (s0[0] + s1[0] + gp[...]) + b2[...]
        o[...] = jnp.dot(g, wfc[...], preferred_element_type=jnp.float32) + bfc[...]

    return pl.pallas_call(
        body,
        grid=grid,
        in_specs=s_spec + [pl.BlockSpec((bn, d), lambda i: (i, 0))] + d_spec + [
            pl.BlockSpec((1, d), lambda i: (0, 0)),
            pl.BlockSpec((d, d), lambda i: (0, 0)),
            pl.BlockSpec((1, d), lambda i: (0, 0)),
        ],
        out_specs=pl.BlockSpec((bn, d), lambda i: (i, 0)),
        out_shape=jax.ShapeDtypeStruct((n, d), jnp.float32),
    )


@jax.jit
def kernel(x, edge_index, W1, b1, W2, b2, Wfc, bfc):
    n, d = x.shape
    e = edge_index.shape[1]
    cpt = _round_up(e, 4 * NB * NW * K) // (NW * K)
    epad = cpt * NW * K
    npad = _round_up(n + 1, NS * 8)
    bn = 1000
    assert n % bn == 0 and npad >= n + 1

    pad = jnp.arange(epad - e, dtype=jnp.int32)
    src = jnp.concatenate(
        [edge_index[0], pad % n]).reshape(NW, cpt, K)
    dst = jnp.concatenate(
        [edge_index[1], n + pad % (npad - n)]).reshape(NW, cpt, K)

    zeros_d = jnp.zeros((npad, d), jnp.float32)
    ones_d = jnp.ones((K, d), jnp.float32)

    t1 = _matmul(n, bn, d)(x, W1)
    dcnt = _make_deg_kernel(cpt, npad, d)(dst, zeros_d, ones_d)
    dinv = _dinv_kernel(npad, d)(dcnt, dcnt)
    hprime = _scale(n, bn, d)(dinv, t1)
    s1 = _make_agg_kernel(cpt, npad, d)(hprime, src, dst, zeros_d)
    gprime = _mid_layer(n, bn, d)(s1, s1, hprime, dinv,
                                  b1.reshape(1, d), W2)
    s2 = _make_agg_kernel(cpt, npad, d)(gprime, src, dst, zeros_d)
    out = _final_layer(n, bn, d)(s2, s2, gprime, dinv,
                                 b2.reshape(1, d), Wfc, bfc.reshape(1, d))
    return out


# speedup vs baseline: 1.0603x; 1.0603x over previous
import functools

import jax
import jax.numpy as jnp
from jax import lax
from jax.experimental import pallas as pl
from jax.experimental.pallas import tpu as pltpu
from jax.experimental.pallas import tpu_sc as plsc

NC = 2
NS = 16
NW = NC * NS
K = 64
NB = 4


def _round_up(a, m):
    return (a + m - 1) // m * m


def _sc_mesh():
    return plsc.VectorSubcoreMesh(
        core_axis_name="c", subcore_axis_name="s", num_cores=NC, num_subcores=NS
    )


def _fill(ref, val, d):
    def row(r, carry):
        for l in range(d // 16):
            ref[r, pl.ds(16 * l, 16)] = jnp.full((16,), val, jnp.float32)
        return carry
    lax.fori_loop(0, K, row, 0)


def _zero_acc_slice(buf, acc, base, rpt, sem):
    descs = [pltpu.async_copy(buf, acc.at[pl.ds(base + i * K, K)], sem)
             for i in range(rpt // K)]
    if rpt % K:
        descs.append(pltpu.async_copy(
            buf.at[pl.ds(0, rpt % K)],
            acc.at[pl.ds(base + (rpt // K) * K, rpt % K)], sem))
    for desc in descs:
        desc.wait()


def _make_deg_kernel(cpt, npad, d):
    rpt = npad // NS

    def body(dst_hbm, out_hbm, dst_v, ones_v, acc, sem):
        cid = lax.axis_index("c")
        sid = lax.axis_index("s")
        wid = sid * NC + cid
        _fill(ones_v, 0.0, d)
        _zero_acc_slice(ones_v, acc, sid * rpt, rpt, sem)
        _fill(ones_v, 1.0, d)
        pltpu.sync_copy(dst_hbm.at[wid], dst_v)
        plsc.subcore_barrier()

        def group(g, carry):
            descs = [
                pltpu.async_copy(ones_v, acc.at[dst_v.at[4 * g + b]], sem,
                                 add=True)
                for b in range(4)
            ]
            for desc in descs:
                desc.wait()
            return carry

        lax.fori_loop(0, cpt // 4, group, 0)

        def chunk(j, carry):
            pltpu.sync_copy(ones_v, acc.at[dst_v.at[j]], add=True)
            return carry

        lax.fori_loop(cpt // 4 * 4, cpt, chunk, 0)
        plsc.subcore_barrier()
        pltpu.sync_copy(acc.at[pl.ds(sid * rpt, rpt)],
                        out_hbm.at[cid].at[pl.ds(sid * rpt, rpt)])

    return pl.kernel(
        body,
        out_type=jax.ShapeDtypeStruct((NC, npad, d), jnp.float32),
        mesh=_sc_mesh(),
        scratch_types=[
            pltpu.VMEM((cpt, K), jnp.int32),
            pltpu.VMEM((K, d), jnp.float32),
            pltpu.VMEM_SHARED((npad, d), jnp.float32),
            pltpu.SemaphoreType.DMA,
        ],
    )


def _make_agg_kernel(cpt, npad, d):
    rpt = npad // NS
    assert cpt % (4 * NB) == 0

    def body(h_hbm, src_hbm, dst_hbm, out_hbm,
             src_v, dst_v, r0, r1, r2, r3, acc,
             g0, g1, g2, g3, s0, s1, s2, s3):
        rows = [r0, r1, r2, r3]
        gs = [g0, g1, g2, g3]
        ss = [s0, s1, s2, s3]
        cid = lax.axis_index("c")
        sid = lax.axis_index("s")
        wid = sid * NC + cid
        _fill(r0, 0.0, d)
        _zero_acc_slice(r0, acc, sid * rpt, rpt, g0)
        plsc.subcore_barrier()

        hcpt = cpt // 4
        for half in range(4):
            pltpu.sync_copy(src_hbm.at[wid].at[pl.ds(half * hcpt, hcpt)], src_v)
            pltpu.sync_copy(dst_hbm.at[wid].at[pl.ds(half * hcpt, hcpt)], dst_v)
            for b in range(NB):
                pltpu.async_copy(h_hbm.at[src_v.at[b]], rows[b], gs[b])

            def group(g, carry):
                sds = []
                for b in range(NB):
                    j = NB * g + b
                    pltpu.make_async_copy(h_hbm.at[src_v.at[j]], rows[b],
                                          gs[b]).wait()
                    sds.append(pltpu.async_copy(rows[b], acc.at[dst_v.at[j]],
                                                ss[b], add=True))
                for b in range(NB):
                    jn = jnp.minimum(NB * g + b + NB, hcpt - 1)
                    sds[b].wait()
                    pltpu.async_copy(h_hbm.at[src_v.at[jn]], rows[b], gs[b])
                return carry

            lax.fori_loop(0, hcpt // NB, group, 0)
            for b in range(NB):
                pltpu.make_async_copy(h_hbm.at[src_v.at[hcpt - 1]], rows[b],
                                      gs[b]).wait()
        plsc.subcore_barrier()
        pltpu.sync_copy(acc.at[pl.ds(sid * rpt, rpt)],
                        out_hbm.at[cid].at[pl.ds(sid * rpt, rpt)])

    return pl.kernel(
        body,
        out_type=jax.ShapeDtypeStruct((NC, npad, d), jnp.float32),
        mesh=_sc_mesh(),
        scratch_types=(
            [pltpu.VMEM((cpt // 4, K), jnp.int32),
             pltpu.VMEM((cpt // 4, K), jnp.int32)]
            + [pltpu.VMEM((K, d), jnp.float32) for _ in range(NB)]
            + [pltpu.VMEM_SHARED((npad, d), jnp.float32)]
            + [pltpu.SemaphoreType.DMA for _ in range(2 * NB)]
        ),
    )


def _tc_grid_specs(n, bn, d, npad, n_deg, n_s):
    del npad
    grid = n // bn
    s_spec = [pl.BlockSpec((1, bn, d), (lambda i, j=j: (j, i, 0)))
              for j in range(n_s)]
    d_spec = [pl.BlockSpec((bn, 8), (lambda i: (i, 0)))
              for _ in range(n_deg)]
    return grid, s_spec, d_spec


def _matmul(n, bn, d):
    def body(x, w, o):
        o[...] = jnp.dot(x[...], w[...], preferred_element_type=jnp.float32)

    return pl.pallas_call(
        body,
        grid=n // bn,
        in_specs=[
            pl.BlockSpec((bn, d), lambda i: (i, 0)),
            pl.BlockSpec((d, d), lambda i: (0, 0)),
        ],
        out_specs=pl.BlockSpec((bn, d), lambda i: (i, 0)),
        out_shape=jax.ShapeDtypeStruct((n, d), jnp.float32),
    )


def _scale(n, bn, d):
    def body(d0, d1, t, o, dv):
        dinv = lax.rsqrt(1.0 + d0[0, :, 0:1] + d1[0, :, 0:1])
        o[...] = t[...] * dinv
        dv[...] = jnp.broadcast_to(dinv, (bn, 8))

    return pl.pallas_call(
        body,
        grid=n // bn,
        in_specs=[
            pl.BlockSpec((1, bn, d), lambda i: (0, i, 0)),
            pl.BlockSpec((1, bn, d), lambda i: (1, i, 0)),
            pl.BlockSpec((bn, d), lambda i: (i, 0)),
        ],
        out_specs=[
            pl.BlockSpec((bn, d), lambda i: (i, 0)),
            pl.BlockSpec((bn, 8), lambda i: (i, 0)),
        ],
        out_shape=[
            jax.ShapeDtypeStruct((n, d), jnp.float32),
            jax.ShapeDtypeStruct((n, 8), jnp.float32),
        ],
    )


def _mid_layer(n, bn, d):
    grid, s_spec, d_spec = _tc_grid_specs(n, bn, d, None, 1, 2)

    def body(s0, s1, hp, dv, b1, w2, o):
        dinv = dv[:, 0:1]
        h = dinv * (s0[0] + s1[0] + hp[...]) + b1[...]
        h = jnp.maximum(h, 0.0)
        o[...] = dinv * jnp.dot(h, w2[...], preferred_element_type=jnp.float32)

    return pl.pallas_call(
        body,
        grid=grid,
        in_specs=s_spec + [pl.BlockSpec((bn, d), lambda i: (i, 0))] + d_spec + [
            pl.BlockSpec((1, d), lambda i: (0, 0)),
            pl.BlockSpec((d, d), lambda i: (0, 0)),
        ],
        out_specs=pl.BlockSpec((bn, d), lambda i: (i, 0)),
        out_shape=jax.ShapeDtypeStruct((n, d), jnp.float32),
    )


def _final_layer(n, bn, d):
    grid, s_spec, d_spec = _tc_grid_specs(n, bn, d, None, 1, 2)

    def body(s0, s1, gp, dv, b2, wfc, bfc, o):
        g = dv[:, 0:1] * (s0[0] + s1[0] + gp[...]) + b2[...]
        o[...] = jnp.dot(g, wfc[...], preferred_element_type=jnp.float32) + bfc[...]

    return pl.pallas_call(
        body,
        grid=grid,
        in_specs=s_spec + [pl.BlockSpec((bn, d), lambda i: (i, 0))] + d_spec + [
            pl.BlockSpec((1, d), lambda i: (0, 0)),
            pl.BlockSpec((d, d), lambda i: (0, 0)),
            pl.BlockSpec((1, d), lambda i: (0, 0)),
        ],
        out_specs=pl.BlockSpec((bn, d), lambda i: (i, 0)),
        out_shape=jax.ShapeDtypeStruct((n, d), jnp.float32),
    )


@jax.jit
def kernel(x, edge_index, W1, b1, W2, b2, Wfc, bfc):
    n, d = x.shape
    e = edge_index.shape[1]
    cpt = _round_up(e, 4 * NB * NW * K) // (NW * K)
    epad = cpt * NW * K
    npad = _round_up(n + 1, NS * 8)
    bn = 1000
    assert n % bn == 0 and npad >= n + 1

    pad = jnp.arange(epad - e, dtype=jnp.int32)
    src = jnp.concatenate(
        [edge_index[0], pad % n]).reshape(NW, cpt, K)
    dst = jnp.concatenate(
        [edge_index[1], n + pad % (npad - n)]).reshape(NW, cpt, K)

    t1 = _matmul(n, bn, d)(x, W1)
    dcnt = _make_deg_kernel(cpt, npad, d)(dst)
    hprime, dinv = _scale(n, bn, d)(dcnt, dcnt, t1)
    s1 = _make_agg_kernel(cpt, npad, d)(hprime, src, dst)
    gprime = _mid_layer(n, bn, d)(s1, s1, hprime, dinv,
                                  b1.reshape(1, d), W2)
    s2 = _make_agg_kernel(cpt, npad, d)(gprime, src, dst)
    out = _final_layer(n, bn, d)(s2, s2, gprime, dinv,
                                 b2.reshape(1, d), Wfc, bfc.reshape(1, d))
    return out


# speedup vs baseline: 1.0634x; 1.0029x over previous
import functools

import jax
import jax.numpy as jnp
from jax import lax
from jax.experimental import pallas as pl
from jax.experimental.pallas import tpu as pltpu
from jax.experimental.pallas import tpu_sc as plsc

NC = 2
NS = 16
NW = NC * NS
K = 64
NB = 4


def _round_up(a, m):
    return (a + m - 1) // m * m


def _sc_mesh():
    return plsc.VectorSubcoreMesh(
        core_axis_name="c", subcore_axis_name="s", num_cores=NC, num_subcores=NS
    )


def _fill(ref, val, d):
    def row(r, carry):
        for l in range(d // 16):
            ref[r, pl.ds(16 * l, 16)] = jnp.full((16,), val, jnp.float32)
        return carry
    lax.fori_loop(0, K, row, 0)


def _zero_acc_slice(buf, acc, base, rpt, sem):
    descs = [pltpu.async_copy(buf, acc.at[pl.ds(base + i * K, K)], sem)
             for i in range(rpt // K)]
    if rpt % K:
        descs.append(pltpu.async_copy(
            buf.at[pl.ds(0, rpt % K)],
            acc.at[pl.ds(base + (rpt // K) * K, rpt % K)], sem))
    for desc in descs:
        desc.wait()


def _make_deg_kernel(cpt, npad, d):
    rpt = npad // NS

    def body(dst_hbm, out_hbm, dst_v, ones_v, acc, sem):
        cid = lax.axis_index("c")
        sid = lax.axis_index("s")
        wid = sid * NC + cid
        _fill(ones_v, 0.0, d)
        _zero_acc_slice(ones_v, acc, sid * rpt, rpt, sem)
        _fill(ones_v, 1.0, d)
        pltpu.sync_copy(dst_hbm.at[wid], dst_v)
        plsc.subcore_barrier()

        def group(g, carry):
            descs = [
                pltpu.async_copy(ones_v, acc.at[dst_v.at[4 * g + b]], sem,
                                 add=True)
                for b in range(4)
            ]
            for desc in descs:
                desc.wait()
            return carry

        lax.fori_loop(0, cpt // 4, group, 0)

        def chunk(j, carry):
            pltpu.sync_copy(ones_v, acc.at[dst_v.at[j]], add=True)
            return carry

        lax.fori_loop(cpt // 4 * 4, cpt, chunk, 0)
        plsc.subcore_barrier()
        pltpu.sync_copy(acc.at[pl.ds(sid * rpt, rpt)],
                        out_hbm.at[cid].at[pl.ds(sid * rpt, rpt)])

    return pl.kernel(
        body,
        out_type=jax.ShapeDtypeStruct((NC, npad, d), jnp.float32),
        mesh=_sc_mesh(),
        scratch_types=[
            pltpu.VMEM((cpt, K), jnp.int32),
            pltpu.VMEM((K, d), jnp.float32),
            pltpu.VMEM_SHARED((npad, d), jnp.float32),
            pltpu.SemaphoreType.DMA,
        ],
    )


def _make_agg_kernel(cpt, npad, d):
    rpt = npad // NS
    assert cpt % (4 * NB) == 0

    def body(h_hbm, src_hbm, dst_hbm, out_hbm,
             src_v, dst_v, r0, r1, r2, r3, acc,
             g0, g1, g2, g3, s0, s1, s2, s3):
        rows = [r0, r1, r2, r3]
        gs = [g0, g1, g2, g3]
        ss = [s0, s1, s2, s3]
        cid = lax.axis_index("c")
        sid = lax.axis_index("s")
        wid = sid * NC + cid
        _fill(r0, 0.0, d)
        _zero_acc_slice(r0, acc, sid * rpt, rpt, g0)
        plsc.subcore_barrier()

        hcpt = cpt // 4
        for half in range(4):
            pltpu.sync_copy(
                src_hbm.at[pl.ds((wid * cpt + half * hcpt) * K, hcpt * K)],
                src_v)
            pltpu.sync_copy(dst_hbm.at[wid].at[pl.ds(half * hcpt, hcpt)], dst_v)
            for b in range(NB):
                pltpu.async_copy(h_hbm.at[src_v.at[pl.ds(b * K, K)]],
                                 rows[b], gs[b])

            def group(g, carry):
                sds = []
                for b in range(NB):
                    j = NB * g + b
                    pltpu.make_async_copy(h_hbm.at[src_v.at[pl.ds(j * K, K)]],
                                          rows[b], gs[b]).wait()
                    sds.append(pltpu.async_copy(rows[b], acc.at[dst_v.at[j]],
                                                ss[b], add=True))
                for b in range(NB):
                    jn = jnp.minimum(NB * g + b + NB, hcpt - 1)
                    sds[b].wait()
                    pltpu.async_copy(h_hbm.at[src_v.at[pl.ds(jn * K, K)]],
                                     rows[b], gs[b])
                return carry

            lax.fori_loop(0, hcpt // NB, group, 0)
            for b in range(NB):
                pltpu.make_async_copy(
                    h_hbm.at[src_v.at[pl.ds((hcpt - 1) * K, K)]],
                    rows[b], gs[b]).wait()
        plsc.subcore_barrier()
        pltpu.sync_copy(acc.at[pl.ds(sid * rpt, rpt)],
                        out_hbm.at[cid].at[pl.ds(sid * rpt, rpt)])

    return pl.kernel(
        body,
        out_type=jax.ShapeDtypeStruct((NC, npad, d), jnp.float32),
        mesh=_sc_mesh(),
        scratch_types=(
            [pltpu.VMEM((cpt // 4 * K,), jnp.int32),
             pltpu.VMEM((cpt // 4, K), jnp.int32)]
            + [pltpu.VMEM((K, d), jnp.float32) for _ in range(NB)]
            + [pltpu.VMEM_SHARED((npad, d), jnp.float32)]
            + [pltpu.SemaphoreType.DMA for _ in range(2 * NB)]
        ),
    )


def _tc_grid_specs(n, bn, d, npad, n_deg, n_s):
    del npad
    grid = n // bn
    s_spec = [pl.BlockSpec((1, bn, d), (lambda i, j=j: (j, i, 0)))
              for j in range(n_s)]
    d_spec = [pl.BlockSpec((bn, 8), (lambda i: (i, 0)))
              for _ in range(n_deg)]
    return grid, s_spec, d_spec


def _matmul(n, bn, d):
    def body(x, w, o):
        o[...] = jnp.dot(x[...], w[...], preferred_element_type=jnp.float32)

    return pl.pallas_call(
        body,
        grid=n // bn,
        in_specs=[
            pl.BlockSpec((bn, d), lambda i: (i, 0)),
            pl.BlockSpec((d, d), lambda i: (0, 0)),
        ],
        out_specs=pl.BlockSpec((bn, d), lambda i: (i, 0)),
        out_shape=jax.ShapeDtypeStruct((n, d), jnp.float32),
    )


def _scale(n, bn, d):
    def body(d0, d1, t, o, dv):
        dinv = lax.rsqrt(1.0 + d0[0, :, 0:1] + d1[0, :, 0:1])
        o[...] = t[...] * dinv
        dv[...] = jnp.broadcast_to(dinv, (bn, 8))

    return pl.pallas_call(
        body,
        grid=n // bn,
        in_specs=[
            pl.BlockSpec((1, bn, d), lambda i: (0, i, 0)),
            pl.BlockSpec((1, bn, d), lambda i: (1, i, 0)),
            pl.BlockSpec((bn, d), lambda i: (i, 0)),
        ],
        out_specs=[
            pl.BlockSpec((bn, d), lambda i: (i, 0)),
            pl.BlockSpec((bn, 8), lambda i: (i, 0)),
        ],
        out_shape=[
            jax.ShapeDtypeStruct((n, d), jnp.float32),
            jax.ShapeDtypeStruct((n, 8), jnp.float32),
        ],
    )


def _mid_layer(n, bn, d):
    grid, s_spec, d_spec = _tc_grid_specs(n, bn, d, None, 1, 2)

    def body(s0, s1, hp, dv, b1, w2, o):
        dinv = dv[:, 0:1]
        h = dinv * (s0[0] + s1[0] + hp[...]) + b1[...]
        h = jnp.maximum(h, 0.0)
        o[...] = dinv * jnp.dot(h, w2[...], preferred_element_type=jnp.float32)

    return pl.pallas_call(
        body,
        grid=grid,
        in_specs=s_spec + [pl.BlockSpec((bn, d), lambda i: (i, 0))] + d_spec + [
            pl.BlockSpec((1, d), lambda i: (0, 0)),
            pl.BlockSpec((d, d), lambda i: (0, 0)),
        ],
        out_specs=pl.BlockSpec((bn, d), lambda i: (i, 0)),
        out_shape=jax.ShapeDtypeStruct((n, d), jnp.float32),
    )


def _final_layer(n, bn, d):
    grid, s_spec, d_spec = _tc_grid_specs(n, bn, d, None, 1, 2)

    def body(s0, s1, gp, dv, b2, wfc, bfc, o):
        g = dv[:, 0:1] * (s0[0] + s1[0] + gp[...]) + b2[...]
        o[...] = jnp.dot(g, wfc[...], preferred_element_type=jnp.float32) + bfc[...]

    return pl.pallas_call(
        body,
        grid=grid,
        in_specs=s_spec + [pl.BlockSpec((bn, d), lambda i: (i, 0))] + d_spec + [
            pl.BlockSpec((1, d), lambda i: (0, 0)),
            pl.BlockSpec((d, d), lambda i: (0, 0)),
            pl.BlockSpec((1, d), lambda i: (0, 0)),
        ],
        out_specs=pl.BlockSpec((bn, d), lambda i: (i, 0)),
        out_shape=jax.ShapeDtypeStruct((n, d), jnp.float32),
    )


@jax.jit
def kernel(x, edge_index, W1, b1, W2, b2, Wfc, bfc):
    n, d = x.shape
    e = edge_index.shape[1]
    cpt = _round_up(e, 4 * NB * NW * K) // (NW * K)
    epad = cpt * NW * K
    npad = _round_up(n + 1, NS * 8)
    bn = 1000
    assert n % bn == 0 and npad >= n + 1

    pad = jnp.arange(epad - e, dtype=jnp.int32)
    src = jnp.concatenate([edge_index[0], pad % n])
    dst = jnp.concatenate(
        [edge_index[1], n + pad % (npad - n)]).reshape(NW, cpt, K)

    t1 = _matmul(n, bn, d)(x, W1)
    dcnt = _make_deg_kernel(cpt, npad, d)(dst)
    hprime, dinv = _scale(n, bn, d)(dcnt, dcnt, t1)
    s1 = _make_agg_kernel(cpt, npad, d)(hprime, src, dst)
    gprime = _mid_layer(n, bn, d)(s1, s1, hprime, dinv,
                                  b1.reshape(1, d), W2)
    s2 = _make_agg_kernel(cpt, npad, d)(gprime, src, dst)
    out = _final_layer(n, bn, d)(s2, s2, gprime, dinv,
                                 b2.reshape(1, d), Wfc, bfc.reshape(1, d))
    return out
